# baseline (speedup 1.0000x reference)
import functools

import jax
import jax.numpy as jnp
from jax import lax
from jax.experimental import pallas as pl
from jax.experimental.pallas import tpu as pltpu
from jax.experimental.pallas import tpu_sc as plsc

_LRADIUS = 128
_RRADIUS = 128
_EMBED_DIM = 768
_T = 16383

_NC = 2
_NS = 16
_NW = _NC * _NS
_B_PAD = 16384
_B_PER_W = _B_PAD // _NW
_CHUNK = 128
_NCHUNK = _B_PER_W // _CHUNK

_mesh = plsc.VectorSubcoreMesh(core_axis_name="c", subcore_axis_name="s")


@functools.partial(
    pl.kernel,
    mesh=_mesh,
    out_type=jax.ShapeDtypeStruct((_T, _EMBED_DIM), jnp.float32),
    scratch_types=[
        pltpu.VMEM((_B_PER_W,), jnp.int32),
        pltpu.VMEM((_CHUNK, _EMBED_DIM), jnp.float32),
        pltpu.VMEM((_NCHUNK, _CHUNK), jnp.int32),
        pltpu.SemaphoreType.DMA,
    ],
)
def _rel_pos_sc(table_hbm, pos_hbm, out_hbm, idx_v, rows_v, widx, sem):
    wid = lax.axis_index("s") * _NC + lax.axis_index("c")
    base_in = wid * _B_PER_W
    is_last = wid == _NW - 1

    pltpu.sync_copy(pos_hbm.at[pl.ds(base_in, _B_PER_W)], idx_v)

    @pl.when(is_last)
    def _():
        lane = lax.iota(jnp.int32, 16)
        for c in range(_NCHUNK):
            for j in range(_CHUNK // 16):
                widx[c, pl.ds(j * 16, 16)] = (
                    (_T - _B_PER_W + c * _CHUNK + j * 16) + lane
                )

    for i in range(_B_PER_W // 16):
        v = idx_v[pl.ds(i * 16, 16)]
        v = jnp.minimum(jnp.maximum(v, -_LRADIUS), _RRADIUS) + _LRADIUS
        idx_v[pl.ds(i * 16, 16)] = v

    for c in range(_NCHUNK):
        pltpu.async_copy(
            table_hbm.at[idx_v.at[pl.ds(c * _CHUNK, _CHUNK)]], rows_v, sem
        ).wait()

        @pl.when(jnp.logical_not(is_last))
        def _():
            row0 = pl.multiple_of(base_in + c * _CHUNK, _CHUNK)
            pltpu.sync_copy(rows_v, out_hbm.at[pl.ds(row0, _CHUNK)])

        @pl.when(is_last)
        def _():
            pltpu.async_copy(rows_v, out_hbm.at[widx.at[c]], sem).wait()


def kernel(position, table):
    pos = position.astype(jnp.int32)
    pos_flat = jnp.concatenate([pos[: _B_PAD - _B_PER_W], pos[_T - _B_PER_W :]])
    return _rel_pos_sc(table, pos_flat)
